# Initial kernel scaffold; baseline (speedup 1.0000x reference)
import jax
import jax.numpy as jnp
from jax.experimental import pallas as pl


def kernel(x, W1, b1, Wc, bc, Wr, br, img_size):
    raise NotImplementedError("write your pallas kernel here")


# speedup vs baseline: 12.2249x; 12.2249x over previous
import functools

import numpy as np

import jax
import jax.numpy as jnp
from jax import lax
from jax.experimental import pallas as pl
from jax.experimental.pallas import tpu as pltpu

_B, _CIN, _CMID = 4, 256, 256
_H = _W = 32
_P = _H * _W
_A = 9
_N = _A * _P
_ROWS = _N // 128
_N_PRE = 6000
_N_POST = 300
_NMS_T = 0.7
_MIN_SIZE = 16.0
_STRIDE = 16
_PAD = 64
_BIG = np.int32(1 << 30)
_INT_MIN = np.int32(-(1 << 31))


def _make_anchors():
    base = float(_STRIDE)
    ratios = [0.5, 1.0, 2.0]
    scales = [8, 16, 32]
    py = px = base / 2.0
    ab = np.zeros((9, 4), np.float32)
    for i, r in enumerate(ratios):
        for j, s in enumerate(scales):
            h = base * s * np.sqrt(r)
            w = base * s * np.sqrt(1.0 / r)
            k = i * 3 + j
            ab[k] = [py - h / 2.0, px - w / 2.0, py + h / 2.0, px + w / 2.0]
    sy = np.arange(_H, dtype=np.float32) * np.float32(_STRIDE)
    sx = np.arange(_W, dtype=np.float32) * np.float32(_STRIDE)
    gx, gy = np.meshgrid(sx, sy)
    shift = np.stack([gy.ravel(), gx.ravel(), gy.ravel(), gx.ravel()], axis=1)
    anc = (shift[:, None, :].astype(np.float32) + ab[None, :, :]).reshape(-1, 4)
    return anc


_ANCHORS_NP = _make_anchors()


def _conv_kernel(x_ref, w_ref, b1_ref, wh_ref, bh_ref, out_ref):
    x = x_ref[0]
    col = lax.broadcasted_iota(jnp.int32, (_CIN, _P), 1)
    cmod = jnp.bitwise_and(col, 31)
    mask_r = (cmod != 31).astype(jnp.float32)
    mask_l = (cmod != 0).astype(jnp.float32)
    acc = jnp.zeros((_CMID, _P), jnp.float32)
    for k in range(9):
        dy, dx = k // 3 - 1, k % 3 - 1
        s = 32 * dy + dx
        xs = x[:, _PAD + s:_PAD + s + _P]
        if dx == 1:
            xs = xs * mask_r
        elif dx == -1:
            xs = xs * mask_l
        acc = acc + lax.dot_general(
            w_ref[k].astype(jnp.bfloat16), xs.astype(jnp.bfloat16),
            (((1,), (0,)), ((), ())),
            preferred_element_type=jnp.float32)
    feat = jnp.maximum(acc + b1_ref[...], 0.0)
    head = lax.dot_general(
        wh_ref[...].astype(jnp.bfloat16), feat.astype(jnp.bfloat16),
        (((1,), (0,)), ((), ())),
        preferred_element_type=jnp.float32) + bh_ref[...]
    out_ref[0] = head


def _propose_kernel(l0_ref, l1_ref, t0_ref, t1_ref, t2_ref, t3_ref,
                    ay1_ref, ax1_ref, ay2_ref, ax2_ref, img_ref, out_ref):
    f32 = jnp.float32
    l0 = l0_ref[...]
    l1 = l1_ref[...]
    m = jnp.maximum(l0, l1)
    e0 = jnp.exp(l0 - m)
    e1 = jnp.exp(l1 - m)
    fg = e1 / (e0 + e1)

    ay1 = ay1_ref[...]
    ax1 = ax1_ref[...]
    ay2 = ay2_ref[...]
    ax2 = ax2_ref[...]
    ah = ay2 - ay1
    aw = ax2 - ax1
    acy = ay1 + 0.5 * ah
    acx = ax1 + 0.5 * aw
    dy = t0_ref[...]
    dx = t1_ref[...]
    dh = t2_ref[...]
    dw = t3_ref[...]
    ncy = dy * ah + acy
    ncx = dx * aw + acx
    nh = jnp.exp(dh) * ah
    nw = jnp.exp(dw) * aw
    img_h = img_ref[0, 0]
    img_w = img_ref[0, 1]
    y1 = jnp.clip(ncy - 0.5 * nh, 0.0, img_h)
    x1 = jnp.clip(ncx - 0.5 * nw, 0.0, img_w)
    y2 = jnp.clip(ncy + 0.5 * nh, 0.0, img_h)
    x2 = jnp.clip(ncx + 0.5 * nw, 0.0, img_w)
    hs = y2 - y1
    ws = x2 - x1
    area = hs * ws
    valid = (hs >= _MIN_SIZE) & (ws >= _MIN_SIZE)
    neg_inf = f32(-jnp.inf)
    sc = jnp.where(valid, fg, neg_inf)

    bits = lax.bitcast_convert_type(sc, jnp.int32)
    key = jnp.where(bits < 0, bits ^ np.int32(0x7FFFFFFF), bits)

    def count_ge(th):
        return jnp.sum((key >= th).astype(jnp.int32), axis=(1, 2), keepdims=True)

    cnt = count_ge(np.int32(0))
    prefix = jnp.where(cnt >= _N_PRE, np.int32(0), _INT_MIN)
    for b in range(30, -1, -1):
        cand_th = prefix | np.int32(1 << b)
        cnt = count_ge(cand_th)
        prefix = jnp.where(cnt >= _N_PRE, cand_th, prefix)
    v_star = prefix
    gt = key > v_star
    n_gt = jnp.sum(gt.astype(jnp.int32), axis=(1, 2), keepdims=True)
    quota = (_N_PRE - n_gt).astype(f32)
    eq = (key == v_star)
    eqf = eq.astype(f32).reshape(_B * _ROWS, 128)
    lane_i = lax.broadcasted_iota(jnp.int32, (128, 128), 0)
    lane_j = lax.broadcasted_iota(jnp.int32, (128, 128), 1)
    lower_lane = (lane_i < lane_j).astype(f32)
    lane_pre = lax.dot_general(
        eqf, lower_lane, (((1,), (0,)), ((), ())),
        preferred_element_type=f32,
        precision=lax.Precision.HIGHEST).reshape(_B, _ROWS, 128)
    row_tot = jnp.sum(eqf.reshape(_B, _ROWS, 128), axis=2)
    row_i = lax.broadcasted_iota(jnp.int32, (_ROWS, _ROWS), 0)
    row_j = lax.broadcasted_iota(jnp.int32, (_ROWS, _ROWS), 1)
    lower_row = (row_i < row_j).astype(f32)
    row_pre = lax.dot_general(
        row_tot, lower_row, (((1,), (0,)), ((), ())),
        preferred_element_type=f32,
        precision=lax.Precision.HIGHEST)
    eq_rank = lane_pre + row_pre[:, :, None]
    in_set = gt | (eq & (eq_rank < quota))

    ridx = (lax.broadcasted_iota(jnp.int32, (_ROWS, 128), 0) * 128
            + lax.broadcasted_iota(jnp.int32, (_ROWS, 128), 1))
    bidx = lax.broadcasted_iota(jnp.int32, (_B, 128), 0).astype(f32)
    lane = lax.broadcasted_iota(jnp.int32, (_B, 128), 1)

    def body(i, alive):
        alive_b = alive > 0.5
        scm = jnp.where(alive_b, sc, neg_inf)
        mmax = jnp.max(scm, axis=(1, 2), keepdims=True)
        hit = alive_b & (scm == mmax)
        cand = jnp.min(jnp.where(hit, ridx, _BIG), axis=(1, 2), keepdims=True)
        found = cand < _BIG
        onehot = ridx == cand

        def pick(v):
            return jnp.sum(jnp.where(onehot, v, 0.0), axis=(1, 2), keepdims=True)

        by1 = pick(y1)
        bx1 = pick(x1)
        by2 = pick(y2)
        bx2 = pick(x2)
        barea = (by2 - by1) * (bx2 - bx1)
        yy1 = jnp.maximum(by1, y1)
        xx1 = jnp.maximum(bx1, x1)
        yy2 = jnp.minimum(by2, y2)
        xx2 = jnp.minimum(bx2, x2)
        inter = (jnp.maximum(yy2 - yy1, 0.0) * jnp.maximum(xx2 - xx1, 0.0))
        iou = inter / (barea + area - inter + 1e-9)
        sup = (iou >= _NMS_T) | onehot
        alive = alive * (1.0 - (sup & found).astype(f32))

        foundf = found.astype(f32)
        c1 = (by1 * foundf)[:, 0, :]
        c2 = (bx1 * foundf)[:, 0, :]
        c3 = (by2 * foundf)[:, 0, :]
        c4 = (bx2 * foundf)[:, 0, :]
        tile = jnp.where(lane == 0, bidx, 0.0)
        tile = jnp.where(lane == 1, c1, tile)
        tile = jnp.where(lane == 2, c2, tile)
        tile = jnp.where(lane == 3, c3, tile)
        tile = jnp.where(lane == 4, c4, tile)
        out_ref[pl.ds(i, 1)] = tile.reshape(1, _B, 128)
        return alive

    lax.fori_loop(0, _N_POST, body, in_set.astype(f32), unroll=False)


@jax.jit
def kernel(x, W1, b1, Wc, bc, Wr, br, img_size):
    n = x.shape[0]
    f32 = jnp.float32
    x2 = x.reshape(n, _CIN, _P)
    x2p = jnp.pad(x2, ((0, 0), (0, 0), (_PAD, _PAD)))
    wstk = W1.transpose(2, 3, 0, 1).reshape(9, _CMID, _CIN)
    whead = jnp.concatenate([Wr.reshape(-1, _CMID), Wc.reshape(-1, _CMID)], axis=0)
    bhead = jnp.concatenate([br, bc], axis=0).reshape(-1, 1)
    b1c = b1.reshape(_CMID, 1)
    nloc = Wr.shape[0]
    nhead = nloc + Wc.shape[0]

    head = pl.pallas_call(
        _conv_kernel,
        grid=(n,),
        in_specs=[
            pl.BlockSpec((1, _CIN, _P + 2 * _PAD), lambda i: (i, 0, 0)),
            pl.BlockSpec((9, _CMID, _CIN), lambda i: (0, 0, 0)),
            pl.BlockSpec((_CMID, 1), lambda i: (0, 0)),
            pl.BlockSpec((nhead, _CMID), lambda i: (0, 0)),
            pl.BlockSpec((nhead, 1), lambda i: (0, 0)),
        ],
        out_specs=pl.BlockSpec((1, nhead, _P), lambda i: (i, 0, 0)),
        out_shape=jax.ShapeDtypeStruct((n, nhead, _P), f32),
    )(x2p, wstk, b1c, whead, bhead)

    rpn_locs_o = head[:, :nloc, :].reshape(n, _N, 4)
    rpn_scores_o = head[:, nloc:, :].reshape(n, _N, 2)

    s5 = head[:, nloc:, :].reshape(n, _A, _H, _W, 2)
    l0 = s5[..., 0].reshape(n, _ROWS, 128)
    l1 = s5[..., 1].reshape(n, _ROWS, 128)
    locr = head[:, :nloc, :].reshape(n, _N, 4)
    t0 = locr[..., 0].reshape(n, _ROWS, 128)
    t1 = locr[..., 1].reshape(n, _ROWS, 128)
    t2 = locr[..., 2].reshape(n, _ROWS, 128)
    t3 = locr[..., 3].reshape(n, _ROWS, 128)
    anc = jnp.asarray(_ANCHORS_NP)
    ay1 = anc[:, 0].reshape(_ROWS, 128)
    ax1 = anc[:, 1].reshape(_ROWS, 128)
    ay2 = anc[:, 2].reshape(_ROWS, 128)
    ax2 = anc[:, 3].reshape(_ROWS, 128)
    img = jnp.zeros((1, 128), f32).at[0, :2].set(img_size.astype(f32))

    rois_raw = pl.pallas_call(
        _propose_kernel,
        out_shape=jax.ShapeDtypeStruct((_N_POST, n, 128), f32),
    )(l0, l1, t0, t1, t2, t3, ay1, ax1, ay2, ax2, img)

    rois5 = rois_raw[:, :, :5].transpose(1, 0, 2).reshape(n * _N_POST, 5)
    roi_indices = jnp.repeat(jnp.arange(n, dtype=jnp.int32), _N_POST)
    return (rpn_locs_o, rpn_scores_o, rois5, roi_indices, anc)
